# baseline (speedup 1.0000x reference)
import functools

import jax
import jax.numpy as jnp
from jax import lax
from jax.experimental import pallas as pl
from jax.experimental.pallas import tpu as pltpu
from jax.experimental.pallas import tpu_sc as plsc

B = 16
L = 128
LL = L * L
LLL = L * L * L
NPTS = B * LL
VOL_N = B * 2 * LLL

_info = plsc.get_sparse_core_info()
NC = _info.num_cores
NS = _info.num_subcores
NL = _info.num_lanes
NW = NC * NS
PPW = NPTS // NW
P = 512
NCH = PPW // P
NG = P // NL

_mesh = plsc.VectorSubcoreMesh(core_axis_name="c", subcore_axis_name="s")


@functools.partial(
    pl.kernel,
    mesh=_mesh,
    out_type=jax.ShapeDtypeStruct((B * 2 * LL,), jnp.float32),
    scratch_types=[
        pltpu.VMEM((P,), jnp.float32),
        pltpu.VMEM((P,), jnp.float32),
        pltpu.VMEM((P,), jnp.float32),
        pltpu.VMEM((P,), jnp.float32),
        pltpu.VMEM((P,), jnp.float32),
        pltpu.VMEM((P,), jnp.float32),
        pltpu.VMEM((8 * P,), jnp.int32),
        pltpu.VMEM((8 * P,), jnp.int32),
        pltpu.VMEM((8 * P,), jnp.int32),
        pltpu.VMEM((8 * P,), jnp.int32),
        pltpu.VMEM((8 * P,), jnp.float32),
        pltpu.VMEM((8 * P,), jnp.float32),
        pltpu.VMEM((8 * P,), jnp.float32),
        pltpu.VMEM((8 * P,), jnp.float32),
        pltpu.VMEM((2 * P,), jnp.float32),
        pltpu.SemaphoreType.DMA,
        pltpu.SemaphoreType.DMA,
        pltpu.SemaphoreType.DMA,
        pltpu.SemaphoreType.DMA,
    ],
)
def _trilerp_sc(vol_hbm, gx_hbm, gy_hbm, gz_hbm, out_hbm,
                gxv0, gyv0, gzv0, gxv1, gyv1, gzv1,
                idxa0, idxb0, idxa1, idxb1,
                gva0, gvb0, gva1, gvb1, ov,
                sema0, semb0, sema1, semb1):
    wid = lax.axis_index("s") * NC + lax.axis_index("c")
    base_pt = wid * PPW
    gxs = (gxv0, gxv1)
    gys = (gyv0, gyv1)
    gzs = (gzv0, gzv1)
    idxas = (idxa0, idxa1)
    idxbs = (idxb0, idxb1)
    gvas = (gva0, gva1)
    gvbs = (gvb0, gvb1)
    semas = (sema0, sema1)
    sembs = (semb0, semb1)

    def stage(p, off):
        pltpu.sync_copy(gx_hbm.at[pl.ds(off, P)], gxs[p])
        pltpu.sync_copy(gy_hbm.at[pl.ds(off, P)], gys[p])
        pltpu.sync_copy(gz_hbm.at[pl.ds(off, P)], gzs[p])

    def compute_idx(p, off):
        b = off // LL
        vb = b * 2 * LLL
        gxp, gyp, gzp = gxs[p], gys[p], gzs[p]
        idxap, idxbp = idxas[p], idxbs[p]

        def idx_body(g, c2):
            s = g * NL
            x = gxp[pl.ds(s, NL)]
            y = gyp[pl.ds(s, NL)]
            z = gzp[pl.ds(s, NL)]
            x0 = x.astype(jnp.int32)
            y0 = y.astype(jnp.int32)
            z0 = z.astype(jnp.int32)
            x0 = jnp.where(x0 >= L - 1, x0 - (L - 1), x0)
            y0 = jnp.where(y0 >= L - 1, y0 - (L - 1), y0)
            z0 = jnp.where(z0 >= L - 1, z0 - (L - 1), z0)
            x1 = x0 + 1
            y1 = y0 + 1
            z1 = z0 + 1
            x1 = jnp.where(x1 >= L - 1, x1 - (L - 1), x1)
            y1 = jnp.where(y1 >= L - 1, y1 - (L - 1), y1)
            z1 = jnp.where(z1 >= L - 1, z1 - (L - 1), z1)
            yx00 = vb + (y0 * L + x0) * L
            yx01 = vb + (y0 * L + x1) * L
            yx10 = vb + (y1 * L + x0) * L
            yx11 = vb + (y1 * L + x1) * L
            idxap[pl.ds(0 * P + s, NL)] = yx00 + z0
            idxap[pl.ds(1 * P + s, NL)] = yx00 + z0 + LLL
            idxap[pl.ds(2 * P + s, NL)] = yx00 + z1
            idxap[pl.ds(3 * P + s, NL)] = yx00 + z1 + LLL
            idxap[pl.ds(4 * P + s, NL)] = yx01 + z0
            idxap[pl.ds(5 * P + s, NL)] = yx01 + z0 + LLL
            idxap[pl.ds(6 * P + s, NL)] = yx01 + z1
            idxap[pl.ds(7 * P + s, NL)] = yx01 + z1 + LLL
            idxbp[pl.ds(0 * P + s, NL)] = yx10 + z0
            idxbp[pl.ds(1 * P + s, NL)] = yx10 + z0 + LLL
            idxbp[pl.ds(2 * P + s, NL)] = yx10 + z1
            idxbp[pl.ds(3 * P + s, NL)] = yx10 + z1 + LLL
            idxbp[pl.ds(4 * P + s, NL)] = yx11 + z0
            idxbp[pl.ds(5 * P + s, NL)] = yx11 + z0 + LLL
            idxbp[pl.ds(6 * P + s, NL)] = yx11 + z1
            idxbp[pl.ds(7 * P + s, NL)] = yx11 + z1 + LLL
            return c2

        lax.fori_loop(0, NG, idx_body, 0)

    def start_gathers(p):
        ca = pltpu.async_copy(vol_hbm.at[idxas[p]], gvas[p], semas[p])
        cb = pltpu.async_copy(vol_hbm.at[idxbs[p]], gvbs[p], sembs[p])
        return (ca, cb)

    def blend_store(p, off):
        b = off // LL
        gxp, gyp, gzp = gxs[p], gys[p], gzs[p]
        gvap, gvbp = gvas[p], gvbs[p]

        def blend_body(g, c2):
            s = g * NL
            x = gxp[pl.ds(s, NL)]
            y = gyp[pl.ds(s, NL)]
            z = gzp[pl.ds(s, NL)]
            dx = x - x.astype(jnp.int32).astype(jnp.float32)
            dy = y - y.astype(jnp.int32).astype(jnp.float32)
            dz = z - z.astype(jnp.int32).astype(jnp.float32)
            ex = 1.0 - dx
            ey = 1.0 - dy
            ez = 1.0 - dz
            for c in (0, 1):
                v000 = gvap[pl.ds((0 + c) * P + s, NL)]
                v001 = gvap[pl.ds((2 + c) * P + s, NL)]
                v100 = gvap[pl.ds((4 + c) * P + s, NL)]
                v101 = gvap[pl.ds((6 + c) * P + s, NL)]
                v010 = gvbp[pl.ds((0 + c) * P + s, NL)]
                v011 = gvbp[pl.ds((2 + c) * P + s, NL)]
                v110 = gvbp[pl.ds((4 + c) * P + s, NL)]
                v111 = gvbp[pl.ds((6 + c) * P + s, NL)]
                cx00 = v000 * ex + v100 * dx
                cx10 = v010 * ex + v110 * dx
                cz0 = cx00 * ey + cx10 * dy
                cx01 = v001 * ex + v101 * dx
                cx11 = v011 * ex + v111 * dx
                cz1 = cx01 * ey + cx11 * dy
                ov[pl.ds(c * P + s, NL)] = cz0 * ez + cz1 * dz
            return c2

        lax.fori_loop(0, NG, blend_body, 0)

        q0 = off - b * LL
        ob = b * 2 * LL
        pltpu.sync_copy(ov.at[pl.ds(0, P)], out_hbm.at[pl.ds(ob + q0, P)])
        pltpu.sync_copy(ov.at[pl.ds(P, P)], out_hbm.at[pl.ds(ob + LL + q0, P)])

    copies = [None, None]
    stage(0, base_pt)
    compute_idx(0, base_pt)
    copies[0] = start_gathers(0)
    for ch in range(1, NCH):
        p = ch % 2
        q = 1 - p
        off = base_pt + ch * P
        stage(p, off)
        compute_idx(p, off)
        copies[p] = start_gathers(p)
        copies[q][0].wait()
        copies[q][1].wait()
        blend_store(q, off - P)
    lastp = (NCH - 1) % 2
    copies[lastp][0].wait()
    copies[lastp][1].wait()
    blend_store(lastp, base_pt + (NCH - 1) * P)


def kernel(volume, grid):
    vol_flat = volume.reshape(VOL_N)
    g = grid.reshape(NPTS, 3)
    gx = g[:, 0]
    gy = g[:, 1]
    gz = g[:, 2]
    out = _trilerp_sc(vol_flat, gx, gy, gz)
    return out.reshape(B, 2, L, L)


# speedup vs baseline: 1.0062x; 1.0062x over previous
import functools

import jax
import jax.numpy as jnp
from jax import lax
from jax.experimental import pallas as pl
from jax.experimental.pallas import tpu as pltpu
from jax.experimental.pallas import tpu_sc as plsc

B = 16
L = 128
LL = L * L
LLL = L * L * L
NPTS = B * LL
VOL_N = B * 2 * LLL

_info = plsc.get_sparse_core_info()
NC = _info.num_cores
NS = _info.num_subcores
NL = _info.num_lanes
NW = NC * NS
PPW = NPTS // NW
P = 1024
NCH = PPW // P
NG = P // NL

_mesh = plsc.VectorSubcoreMesh(core_axis_name="c", subcore_axis_name="s")


@functools.partial(
    pl.kernel,
    mesh=_mesh,
    out_type=jax.ShapeDtypeStruct((B * 2 * LL,), jnp.float32),
    scratch_types=[
        pltpu.VMEM((P,), jnp.float32),
        pltpu.VMEM((P,), jnp.float32),
        pltpu.VMEM((P,), jnp.float32),
        pltpu.VMEM((P,), jnp.float32),
        pltpu.VMEM((P,), jnp.float32),
        pltpu.VMEM((P,), jnp.float32),
        pltpu.VMEM((8 * P,), jnp.int32),
        pltpu.VMEM((8 * P,), jnp.int32),
        pltpu.VMEM((8 * P,), jnp.int32),
        pltpu.VMEM((8 * P,), jnp.int32),
        pltpu.VMEM((8 * P,), jnp.float32),
        pltpu.VMEM((8 * P,), jnp.float32),
        pltpu.VMEM((8 * P,), jnp.float32),
        pltpu.VMEM((8 * P,), jnp.float32),
        pltpu.VMEM((2 * P,), jnp.float32),
        pltpu.SemaphoreType.DMA,
        pltpu.SemaphoreType.DMA,
        pltpu.SemaphoreType.DMA,
        pltpu.SemaphoreType.DMA,
    ],
)
def _trilerp_sc(vol_hbm, gx_hbm, gy_hbm, gz_hbm, out_hbm,
                gxv0, gyv0, gzv0, gxv1, gyv1, gzv1,
                idxa0, idxb0, idxa1, idxb1,
                gva0, gvb0, gva1, gvb1, ov,
                sema0, semb0, sema1, semb1):
    wid = lax.axis_index("s") * NC + lax.axis_index("c")
    base_pt = wid * PPW
    gxs = (gxv0, gxv1)
    gys = (gyv0, gyv1)
    gzs = (gzv0, gzv1)
    idxas = (idxa0, idxa1)
    idxbs = (idxb0, idxb1)
    gvas = (gva0, gva1)
    gvbs = (gvb0, gvb1)
    semas = (sema0, sema1)
    sembs = (semb0, semb1)

    def stage(p, off):
        pltpu.sync_copy(gx_hbm.at[pl.ds(off, P)], gxs[p])
        pltpu.sync_copy(gy_hbm.at[pl.ds(off, P)], gys[p])
        pltpu.sync_copy(gz_hbm.at[pl.ds(off, P)], gzs[p])

    def compute_idx(p, off):
        b = off // LL
        vb = b * 2 * LLL
        gxp, gyp, gzp = gxs[p], gys[p], gzs[p]
        idxap, idxbp = idxas[p], idxbs[p]

        def idx_body(g, c2):
            s = g * NL
            x = gxp[pl.ds(s, NL)]
            y = gyp[pl.ds(s, NL)]
            z = gzp[pl.ds(s, NL)]
            x0 = x.astype(jnp.int32)
            y0 = y.astype(jnp.int32)
            z0 = z.astype(jnp.int32)
            x0 = jnp.where(x0 >= L - 1, x0 - (L - 1), x0)
            y0 = jnp.where(y0 >= L - 1, y0 - (L - 1), y0)
            z0 = jnp.where(z0 >= L - 1, z0 - (L - 1), z0)
            x1 = x0 + 1
            y1 = y0 + 1
            z1 = z0 + 1
            x1 = jnp.where(x1 >= L - 1, x1 - (L - 1), x1)
            y1 = jnp.where(y1 >= L - 1, y1 - (L - 1), y1)
            z1 = jnp.where(z1 >= L - 1, z1 - (L - 1), z1)
            yx00 = vb + (y0 * L + x0) * L
            yx01 = vb + (y0 * L + x1) * L
            yx10 = vb + (y1 * L + x0) * L
            yx11 = vb + (y1 * L + x1) * L
            idxap[pl.ds(0 * P + s, NL)] = yx00 + z0
            idxap[pl.ds(1 * P + s, NL)] = yx00 + z0 + LLL
            idxap[pl.ds(2 * P + s, NL)] = yx00 + z1
            idxap[pl.ds(3 * P + s, NL)] = yx00 + z1 + LLL
            idxap[pl.ds(4 * P + s, NL)] = yx01 + z0
            idxap[pl.ds(5 * P + s, NL)] = yx01 + z0 + LLL
            idxap[pl.ds(6 * P + s, NL)] = yx01 + z1
            idxap[pl.ds(7 * P + s, NL)] = yx01 + z1 + LLL
            idxbp[pl.ds(0 * P + s, NL)] = yx10 + z0
            idxbp[pl.ds(1 * P + s, NL)] = yx10 + z0 + LLL
            idxbp[pl.ds(2 * P + s, NL)] = yx10 + z1
            idxbp[pl.ds(3 * P + s, NL)] = yx10 + z1 + LLL
            idxbp[pl.ds(4 * P + s, NL)] = yx11 + z0
            idxbp[pl.ds(5 * P + s, NL)] = yx11 + z0 + LLL
            idxbp[pl.ds(6 * P + s, NL)] = yx11 + z1
            idxbp[pl.ds(7 * P + s, NL)] = yx11 + z1 + LLL
            return c2

        lax.fori_loop(0, NG, idx_body, 0)

    def start_gathers(p):
        ca = pltpu.async_copy(vol_hbm.at[idxas[p]], gvas[p], semas[p])
        cb = pltpu.async_copy(vol_hbm.at[idxbs[p]], gvbs[p], sembs[p])
        return (ca, cb)

    def blend_store(p, off):
        b = off // LL
        gxp, gyp, gzp = gxs[p], gys[p], gzs[p]
        gvap, gvbp = gvas[p], gvbs[p]

        def blend_body(g, c2):
            s = g * NL
            x = gxp[pl.ds(s, NL)]
            y = gyp[pl.ds(s, NL)]
            z = gzp[pl.ds(s, NL)]
            dx = x - x.astype(jnp.int32).astype(jnp.float32)
            dy = y - y.astype(jnp.int32).astype(jnp.float32)
            dz = z - z.astype(jnp.int32).astype(jnp.float32)
            ex = 1.0 - dx
            ey = 1.0 - dy
            ez = 1.0 - dz
            for c in (0, 1):
                v000 = gvap[pl.ds((0 + c) * P + s, NL)]
                v001 = gvap[pl.ds((2 + c) * P + s, NL)]
                v100 = gvap[pl.ds((4 + c) * P + s, NL)]
                v101 = gvap[pl.ds((6 + c) * P + s, NL)]
                v010 = gvbp[pl.ds((0 + c) * P + s, NL)]
                v011 = gvbp[pl.ds((2 + c) * P + s, NL)]
                v110 = gvbp[pl.ds((4 + c) * P + s, NL)]
                v111 = gvbp[pl.ds((6 + c) * P + s, NL)]
                cx00 = v000 * ex + v100 * dx
                cx10 = v010 * ex + v110 * dx
                cz0 = cx00 * ey + cx10 * dy
                cx01 = v001 * ex + v101 * dx
                cx11 = v011 * ex + v111 * dx
                cz1 = cx01 * ey + cx11 * dy
                ov[pl.ds(c * P + s, NL)] = cz0 * ez + cz1 * dz
            return c2

        lax.fori_loop(0, NG, blend_body, 0)

        q0 = off - b * LL
        ob = b * 2 * LL
        pltpu.sync_copy(ov.at[pl.ds(0, P)], out_hbm.at[pl.ds(ob + q0, P)])
        pltpu.sync_copy(ov.at[pl.ds(P, P)], out_hbm.at[pl.ds(ob + LL + q0, P)])

    copies = [None, None]
    stage(0, base_pt)
    compute_idx(0, base_pt)
    copies[0] = start_gathers(0)
    for ch in range(1, NCH):
        p = ch % 2
        q = 1 - p
        off = base_pt + ch * P
        stage(p, off)
        compute_idx(p, off)
        copies[p] = start_gathers(p)
        copies[q][0].wait()
        copies[q][1].wait()
        blend_store(q, off - P)
    lastp = (NCH - 1) % 2
    copies[lastp][0].wait()
    copies[lastp][1].wait()
    blend_store(lastp, base_pt + (NCH - 1) * P)


def kernel(volume, grid):
    vol_flat = volume.reshape(VOL_N)
    g = grid.reshape(NPTS, 3)
    gx = g[:, 0]
    gy = g[:, 1]
    gz = g[:, 2]
    out = _trilerp_sc(vol_flat, gx, gy, gz)
    return out.reshape(B, 2, L, L)
